# Initial kernel scaffold; baseline (speedup 1.0000x reference)
import jax
import jax.numpy as jnp
from jax.experimental import pallas as pl


def kernel(nodes, edges, edge2seq, ops, mask, entity_table, op_table, gnn_w, gnn_wih, gnn_whh, gnn_bih, gnn_bhh, gru_wih, gru_whh, gru_bih, gru_bhh, w1, b1, w2, b2, w3, b3, wt, bt):
    raise NotImplementedError("write your pallas kernel here")


# speedup vs baseline: 3.5894x; 3.5894x over previous
import functools

import jax
import jax.numpy as jnp
from jax import lax
from jax.experimental import pallas as pl
from jax.experimental.pallas import tpu as pltpu
from jax.experimental.pallas import tpu_sc as plsc

HID = 128
N_NODES = 10000
N_EDGES = 320000
N_ITEM = 10000
B = 1024
L = 50
NC = 2
NS = 16
NW = NC * NS

_MESH = plsc.VectorSubcoreMesh(core_axis_name="c", subcore_axis_name="s")


def _sc_gather(table, idx, chunk):
  n_idx = idx.shape[0]
  per_w = n_idx // NW
  n_chunks = per_w // chunk
  assert per_w % chunk == 0 and chunk % 8 == 0 and chunk <= 128

  @functools.partial(
      pl.kernel,
      out_type=jax.ShapeDtypeStruct((n_idx, HID), jnp.float32),
      mesh=_MESH,
      scratch_types=[
          pltpu.VMEM((chunk,), jnp.int32),
          pltpu.VMEM((chunk, HID), jnp.float32),
          pltpu.SemaphoreType.DMA,
      ],
  )
  def k(table_hbm, idx_hbm, out_hbm, idx_v, rows_v, sem):
    wid = lax.axis_index("c") * NS + lax.axis_index("s")
    base = wid * per_w

    def body(i, carry):
      off = base + i * chunk
      pltpu.sync_copy(idx_hbm.at[pl.ds(off, chunk)], idx_v)
      pltpu.async_copy(table_hbm.at[idx_v], rows_v, sem).wait()
      pltpu.sync_copy(rows_v, out_hbm.at[pl.ds(off, chunk)])
      return carry

    lax.fori_loop(0, n_chunks, body, 0)

  return k(table, idx)


def _sc_edge_scatter_add(m, src, dst, zeros):
  chunk = 80
  per_w = N_EDGES // NW
  n_chunks = per_w // chunk
  n_pad = 10240
  rows_per_tile = n_pad // NS

  @functools.partial(
      pl.kernel,
      out_type=jax.ShapeDtypeStruct((NC, n_pad, HID), jnp.float32),
      mesh=_MESH,
      scratch_types=[
          pltpu.VMEM((chunk,), jnp.int32),
          pltpu.VMEM((chunk,), jnp.int32),
          pltpu.VMEM((chunk, HID), jnp.float32),
          pltpu.VMEM_SHARED((n_pad, HID), jnp.float32),
          pltpu.SemaphoreType.DMA,
      ],
  )
  def k(m_hbm, src_hbm, dst_hbm, zero_hbm, out_hbm,
        sidx_v, didx_v, rows_v, acc_sh, sem):
    c = lax.axis_index("c")
    s = lax.axis_index("s")
    wid = c * NS + s
    pltpu.sync_copy(zero_hbm.at[pl.ds(s * rows_per_tile, rows_per_tile)],
                    acc_sh.at[pl.ds(s * rows_per_tile, rows_per_tile)])
    plsc.subcore_barrier()

    base = wid * per_w

    def body(i, carry):
      off = base + i * chunk
      pltpu.sync_copy(src_hbm.at[pl.ds(off, chunk)], sidx_v)
      pltpu.sync_copy(dst_hbm.at[pl.ds(off, chunk)], didx_v)
      pltpu.async_copy(m_hbm.at[sidx_v], rows_v, sem).wait()
      pltpu.sync_copy(rows_v, acc_sh.at[didx_v], add=True)
      return carry

    lax.fori_loop(0, n_chunks, body, 0)
    plsc.subcore_barrier()
    pltpu.sync_copy(acc_sh.at[pl.ds(s * rows_per_tile, rows_per_tile)],
                    out_hbm.at[c, pl.ds(s * rows_per_tile, rows_per_tile)])

  return k(m, src, dst, zeros)


def _tc_matmul(x, w):
  n = x.shape[0]
  blk = 1000

  def body(x_ref, w_ref, o_ref):
    o_ref[...] = jnp.dot(x_ref[...], w_ref[...],
                         preferred_element_type=jnp.float32)

  return pl.pallas_call(
      body,
      grid=(n // blk,),
      in_specs=[pl.BlockSpec((blk, HID), lambda i: (i, 0)),
                pl.BlockSpec((HID, HID), lambda i: (0, 0))],
      out_specs=pl.BlockSpec((blk, HID), lambda i: (i, 0)),
      out_shape=jax.ShapeDtypeStruct((n, HID), jnp.float32),
  )(x, w)


def _gates(gi, gh, h):
  i_r, i_z, i_n = gi[:, :HID], gi[:, HID:2 * HID], gi[:, 2 * HID:]
  h_r, h_z, h_n = gh[:, :HID], gh[:, HID:2 * HID], gh[:, 2 * HID:]
  r = jax.nn.sigmoid(i_r + h_r)
  z = jax.nn.sigmoid(i_z + h_z)
  n = jnp.tanh(i_n + r * h_n)
  return (1.0 - z) * n + z * h


def _tc_gru_cell(agg2, x, wih_t, whh_t, bih, bhh):
  n = x.shape[0]
  blk = 1000

  def body(agg_ref, x_ref, wih_ref, whh_ref, bih_ref, bhh_ref, o_ref):
    a = agg_ref[0] + agg_ref[1]
    h = x_ref[...]
    gi = jnp.dot(a, wih_ref[...], preferred_element_type=jnp.float32) + bih_ref[0]
    gh = jnp.dot(h, whh_ref[...], preferred_element_type=jnp.float32) + bhh_ref[0]
    o_ref[...] = _gates(gi, gh, h)

  return pl.pallas_call(
      body,
      grid=(n // blk,),
      in_specs=[pl.BlockSpec((NC, blk, HID), lambda i: (0, i, 0)),
                pl.BlockSpec((blk, HID), lambda i: (i, 0)),
                pl.BlockSpec((HID, 3 * HID), lambda i: (0, 0)),
                pl.BlockSpec((HID, 3 * HID), lambda i: (0, 0)),
                pl.BlockSpec((1, 3 * HID), lambda i: (0, 0)),
                pl.BlockSpec((1, 3 * HID), lambda i: (0, 0))],
      out_specs=pl.BlockSpec((blk, HID), lambda i: (i, 0)),
      out_shape=jax.ShapeDtypeStruct((n, HID), jnp.float32),
  )(agg2, x, wih_t, whh_t, bih, bhh)


def _tc_seq_gru(emb_t, wih_t, whh_t, bih, bhh):
  blk = 256

  def body(emb_ref, wih_ref, whh_ref, bih_ref, bhh_ref, o_ref):
    wih = wih_ref[...]
    whh = whh_ref[...]
    bi = bih_ref[0]
    bh = bhh_ref[0]

    def step(t, h):
      xt = emb_ref[t]
      gi = jnp.dot(xt, wih, preferred_element_type=jnp.float32) + bi
      gh = jnp.dot(h, whh, preferred_element_type=jnp.float32) + bh
      h2 = _gates(gi, gh, h)
      o_ref[t] = h2
      return h2

    lax.fori_loop(0, L, step, jnp.zeros((blk, HID), jnp.float32))

  return pl.pallas_call(
      body,
      grid=(B // blk,),
      in_specs=[pl.BlockSpec((L, blk, HID), lambda i: (0, i, 0)),
                pl.BlockSpec((HID, 3 * HID), lambda i: (0, 0)),
                pl.BlockSpec((HID, 3 * HID), lambda i: (0, 0)),
                pl.BlockSpec((1, 3 * HID), lambda i: (0, 0)),
                pl.BlockSpec((1, 3 * HID), lambda i: (0, 0))],
      out_specs=pl.BlockSpec((L, blk, HID), lambda i: (0, i, 0)),
      out_shape=jax.ShapeDtypeStruct((L, B, HID), jnp.float32),
  )(emb_t, wih_t, whh_t, bih, bhh)


def _tc_attention_score(gnn_t, gru_t, mask_t, w1_t, b1, w2_t, b2, w3_t, b3,
                        wt_t, bt, item_t):
  blk = 128

  def body(g_ref, r_ref, m_ref, w1_ref, b1_ref, w2_ref, b2_ref, w3_ref,
           b3_ref, wt_ref, bt_ref, item_ref, o_ref):
    last = jnp.concatenate([g_ref[L - 1], r_ref[L - 1]], axis=-1)
    q = jnp.dot(last, w1_ref[...], preferred_element_type=jnp.float32) + b1_ref[0]
    w2 = w2_ref[...]
    b2v = b2_ref[0]
    w3 = w3_ref[...]
    b3v = b3_ref[0, 0]

    def step(t, carry):
      f_t = jnp.concatenate([g_ref[t], r_ref[t]], axis=-1)
      e_t = jax.nn.sigmoid(
          q + jnp.dot(f_t, w2, preferred_element_type=jnp.float32) + b2v)
      a_t = jnp.dot(e_t, w3, preferred_element_type=jnp.float32) + b3v
      a_t = a_t * (1.0 - m_ref[t][:, None])
      return carry + f_t * a_t

    sg = lax.fori_loop(0, L, step, jnp.zeros((blk, 2 * HID), jnp.float32))
    s = (jnp.dot(last, wt_ref[0], preferred_element_type=jnp.float32)
         + jnp.dot(sg, wt_ref[1], preferred_element_type=jnp.float32)
         + bt_ref[0])
    o_ref[...] = jnp.dot(s, item_ref[...], preferred_element_type=jnp.float32)

  return pl.pallas_call(
      body,
      grid=(B // blk,),
      in_specs=[pl.BlockSpec((L, blk, HID), lambda i: (0, i, 0)),
                pl.BlockSpec((L, blk, HID), lambda i: (0, i, 0)),
                pl.BlockSpec((L, blk), lambda i: (0, i)),
                pl.BlockSpec((2 * HID, 2 * HID), lambda i: (0, 0)),
                pl.BlockSpec((1, 2 * HID), lambda i: (0, 0)),
                pl.BlockSpec((2 * HID, 2 * HID), lambda i: (0, 0)),
                pl.BlockSpec((1, 2 * HID), lambda i: (0, 0)),
                pl.BlockSpec((2 * HID, 1), lambda i: (0, 0)),
                pl.BlockSpec((1, 1), lambda i: (0, 0)),
                pl.BlockSpec((NC, 2 * HID, HID), lambda i: (0, 0, 0)),
                pl.BlockSpec((1, HID), lambda i: (0, 0)),
                pl.BlockSpec((HID, N_ITEM), lambda i: (0, 0))],
      out_specs=pl.BlockSpec((blk, N_ITEM), lambda i: (i, 0)),
      out_shape=jax.ShapeDtypeStruct((B, N_ITEM), jnp.float32),
  )(gnn_t, gru_t, mask_t, w1_t, b1, w2_t, b2, w3_t, b3, wt_t, bt, item_t)


def kernel(nodes, edges, edge2seq, ops, mask, entity_table, op_table, gnn_w,
           gnn_wih, gnn_whh, gnn_bih, gnn_bhh, gru_wih, gru_whh, gru_bih,
           gru_bhh, w1, b1, w2, b2, w3, b3, wt, bt):
  f32 = jnp.float32
  nodes_pad = jnp.concatenate(
      [nodes.astype(jnp.int32),
       jnp.arange(10240 - N_NODES, dtype=jnp.int32)])
  ops_t = jnp.transpose(ops).reshape(-1).astype(jnp.int32)
  e2s_t = jnp.transpose(edge2seq).reshape(-1).astype(jnp.int32)
  src = edges[0].astype(jnp.int32)
  dst = edges[1].astype(jnp.int32)
  zeros = jnp.zeros((10240, HID), f32)
  gnn_wih_t = jnp.transpose(gnn_wih)
  gnn_whh_t = jnp.transpose(gnn_whh)
  gru_wih_t = jnp.transpose(gru_wih)
  gru_whh_t = jnp.transpose(gru_whh)
  w1_t = jnp.transpose(w1)
  w2_t = jnp.transpose(w2)
  w3_t = jnp.transpose(w3)
  wt_t = jnp.transpose(wt).reshape(NC, 2 * HID, HID)
  item_t = jnp.transpose(entity_table[:N_ITEM])
  mask_t = jnp.transpose(mask).astype(f32)
  b2d = lambda v: v.reshape(1, -1)

  x = _sc_gather(entity_table, nodes_pad, 80)[:N_NODES]
  for i in range(2):
    m = _tc_matmul(x, gnn_w[i])
    agg2 = _sc_edge_scatter_add(m, src, dst, zeros)
    x = _tc_gru_cell(agg2, x, gnn_wih_t, gnn_whh_t,
                     b2d(gnn_bih), b2d(gnn_bhh))

  gnn_t = _sc_gather(x, e2s_t, 80).reshape(L, B, HID)

  emb_t = _sc_gather(op_table, ops_t, 80).reshape(L, B, HID)
  gru_t = _tc_seq_gru(emb_t, gru_wih_t, gru_whh_t,
                      b2d(gru_bih), b2d(gru_bhh))

  return _tc_attention_score(gnn_t, gru_t, mask_t, w1_t, b2d(b1), w2_t,
                             b2d(b2), w3_t, b2d(b3).reshape(1, 1), wt_t,
                             b2d(bt), item_t)


# speedup vs baseline: 5.6295x; 1.5684x over previous
import functools

import jax
import jax.numpy as jnp
from jax import lax
from jax.experimental import pallas as pl
from jax.experimental.pallas import tpu as pltpu
from jax.experimental.pallas import tpu_sc as plsc

HID = 128
N_NODES = 10000
N_EDGES = 320000
N_ITEM = 10000
B = 1024
L = 50
NC = 2
NS = 16
NW = NC * NS

_MESH = plsc.VectorSubcoreMesh(core_axis_name="c", subcore_axis_name="s")


def _sc_gather(table, idx, chunk):
  n_idx = idx.shape[0]
  per_w = n_idx // NW
  n_chunks = per_w // chunk
  assert per_w % chunk == 0 and chunk % 8 == 0 and chunk <= 128
  assert n_chunks % 2 == 0 and n_chunks >= 2

  @functools.partial(
      pl.kernel,
      out_type=jax.ShapeDtypeStruct((n_idx, HID), jnp.float32),
      mesh=_MESH,
      scratch_types=[
          pltpu.VMEM((per_w,), jnp.int32),
          pltpu.VMEM((chunk, HID), jnp.float32),
          pltpu.VMEM((chunk, HID), jnp.float32),
          pltpu.SemaphoreType.DMA,
          pltpu.SemaphoreType.DMA,
      ],
  )
  def k(table_hbm, idx_hbm, out_hbm, idx_v, r0, r1, g0, g1):
    wid = lax.axis_index("c") * NS + lax.axis_index("s")
    base = wid * per_w
    pltpu.sync_copy(idx_hbm.at[pl.ds(base, per_w)], idx_v)

    def gather(i, buf, sem):
      pltpu.async_copy(table_hbm.at[idx_v.at[pl.ds(i * chunk, chunk)]],
                       buf, sem)

    def wait0():
      pltpu.make_async_copy(table_hbm.at[pl.ds(0, chunk)], r0, g0).wait()

    def out(i, buf):
      pltpu.sync_copy(buf, out_hbm.at[pl.ds(base + i * chunk, chunk)])

    gather(0, r0, g0)

    def body(i, carry):
      gather(2 * i + 1, r1, g1)
      wait0()
      out(2 * i, r0)
      gather(2 * i + 2, r0, g0)
      pltpu.make_async_copy(table_hbm.at[pl.ds(0, chunk)], r1, g1).wait()
      out(2 * i + 1, r1)
      return carry

    lax.fori_loop(0, n_chunks // 2 - 1, body, 0)
    gather(n_chunks - 1, r1, g1)
    wait0()
    out(n_chunks - 2, r0)
    pltpu.make_async_copy(table_hbm.at[pl.ds(0, chunk)], r1, g1).wait()
    out(n_chunks - 1, r1)

  return k(table, idx)


def _sc_edge_scatter_add(m, src, dst3, zeros):
  chunk = 80
  per_w = N_EDGES // NW
  n_chunks = per_w // chunk
  n_pad = 10240
  rows_per_tile = n_pad // NS

  @functools.partial(
      pl.kernel,
      out_type=jax.ShapeDtypeStruct((NC, n_pad, HID), jnp.float32),
      mesh=_MESH,
      scratch_types=[
          pltpu.VMEM((per_w,), jnp.int32),
          pltpu.VMEM((n_chunks, chunk), jnp.int32),
          pltpu.VMEM((chunk, HID), jnp.float32),
          pltpu.VMEM((chunk, HID), jnp.float32),
          pltpu.VMEM_SHARED((n_pad, HID), jnp.float32),
          pltpu.SemaphoreType.DMA,
          pltpu.SemaphoreType.DMA,
      ],
  )
  def k(m_hbm, src_hbm, dst_hbm, zero_hbm, out_hbm,
        sidx_v, didx_v, r0, r1, acc_sh, g0, g1):
    c = lax.axis_index("c")
    s = lax.axis_index("s")
    wid = c * NS + s
    base = wid * per_w
    pltpu.sync_copy(src_hbm.at[pl.ds(base, per_w)], sidx_v)
    pltpu.sync_copy(dst_hbm.at[wid], didx_v)
    pltpu.sync_copy(zero_hbm.at[pl.ds(s * rows_per_tile, rows_per_tile)],
                    acc_sh.at[pl.ds(s * rows_per_tile, rows_per_tile)])
    plsc.subcore_barrier()

    def gather(i, buf, sem):
      pltpu.async_copy(m_hbm.at[sidx_v.at[pl.ds(i * chunk, chunk)]], buf, sem)

    def scat(i, buf):
      pltpu.sync_copy(buf, acc_sh.at[didx_v.at[i]], add=True)

    def wait(buf, sem):
      pltpu.make_async_copy(m_hbm.at[pl.ds(0, chunk)], buf, sem).wait()

    gather(0, r0, g0)

    def body(i, carry):
      gather(2 * i + 1, r1, g1)
      wait(r0, g0)
      scat(2 * i, r0)
      gather(2 * i + 2, r0, g0)
      wait(r1, g1)
      scat(2 * i + 1, r1)
      return carry

    lax.fori_loop(0, (n_chunks - 1) // 2, body, 0)
    wait(r0, g0)
    scat(n_chunks - 1, r0)

    plsc.subcore_barrier()
    pltpu.sync_copy(acc_sh.at[pl.ds(s * rows_per_tile, rows_per_tile)],
                    out_hbm.at[c, pl.ds(s * rows_per_tile, rows_per_tile)])

  return k(m, src, dst3, zeros)


def _tc_matmul(x, w):
  n = x.shape[0]
  blk = 1000

  def body(x_ref, w_ref, o_ref):
    o_ref[...] = jnp.dot(x_ref[...], w_ref[...],
                         preferred_element_type=jnp.float32)

  return pl.pallas_call(
      body,
      grid=(n // blk,),
      in_specs=[pl.BlockSpec((blk, HID), lambda i: (i, 0)),
                pl.BlockSpec((HID, HID), lambda i: (0, 0))],
      out_specs=pl.BlockSpec((blk, HID), lambda i: (i, 0)),
      out_shape=jax.ShapeDtypeStruct((n, HID), jnp.float32),
  )(x, w)


def _gates(gi, gh, h):
  i_r, i_z, i_n = gi[:, :HID], gi[:, HID:2 * HID], gi[:, 2 * HID:]
  h_r, h_z, h_n = gh[:, :HID], gh[:, HID:2 * HID], gh[:, 2 * HID:]
  r = jax.nn.sigmoid(i_r + h_r)
  z = jax.nn.sigmoid(i_z + h_z)
  n = jnp.tanh(i_n + r * h_n)
  return (1.0 - z) * n + z * h


def _tc_gru_cell(agg2, x, wih_t, whh_t, bih, bhh):
  n = x.shape[0]
  blk = 1000

  def body(agg_ref, x_ref, wih_ref, whh_ref, bih_ref, bhh_ref, o_ref):
    a = agg_ref[0] + agg_ref[1]
    h = x_ref[...]
    gi = jnp.dot(a, wih_ref[...], preferred_element_type=jnp.float32) + bih_ref[0]
    gh = jnp.dot(h, whh_ref[...], preferred_element_type=jnp.float32) + bhh_ref[0]
    o_ref[...] = _gates(gi, gh, h)

  return pl.pallas_call(
      body,
      grid=(n // blk,),
      in_specs=[pl.BlockSpec((NC, blk, HID), lambda i: (0, i, 0)),
                pl.BlockSpec((blk, HID), lambda i: (i, 0)),
                pl.BlockSpec((HID, 3 * HID), lambda i: (0, 0)),
                pl.BlockSpec((HID, 3 * HID), lambda i: (0, 0)),
                pl.BlockSpec((1, 3 * HID), lambda i: (0, 0)),
                pl.BlockSpec((1, 3 * HID), lambda i: (0, 0))],
      out_specs=pl.BlockSpec((blk, HID), lambda i: (i, 0)),
      out_shape=jax.ShapeDtypeStruct((n, HID), jnp.float32),
  )(agg2, x, wih_t, whh_t, bih, bhh)


def _tc_seq_gru(ops_t, tab, wih_t, whh_t, bih, bhh):
  blk = 256
  n_op = tab.shape[0]

  def body(ops_ref, tab_ref, wih_ref, whh_ref, bih_ref, bhh_ref, o_ref):
    table = tab_ref[...]
    wih = wih_ref[...]
    whh = whh_ref[...]
    bi = bih_ref[0]
    bh = bhh_ref[0]

    def step(t, h):
      ids = ops_ref[t]
      oh = (ids[:, None] == lax.broadcasted_iota(jnp.int32, (1, n_op), 1)
            ).astype(jnp.float32)
      xt = jnp.dot(oh, table, preferred_element_type=jnp.float32)
      gi = jnp.dot(xt, wih, preferred_element_type=jnp.float32) + bi
      gh = jnp.dot(h, whh, preferred_element_type=jnp.float32) + bh
      h2 = _gates(gi, gh, h)
      o_ref[t] = h2
      return h2

    lax.fori_loop(0, L, step, jnp.zeros((blk, HID), jnp.float32))

  return pl.pallas_call(
      body,
      grid=(B // blk,),
      in_specs=[pl.BlockSpec((L, blk), lambda i: (0, i)),
                pl.BlockSpec((n_op, HID), lambda i: (0, 0)),
                pl.BlockSpec((HID, 3 * HID), lambda i: (0, 0)),
                pl.BlockSpec((HID, 3 * HID), lambda i: (0, 0)),
                pl.BlockSpec((1, 3 * HID), lambda i: (0, 0)),
                pl.BlockSpec((1, 3 * HID), lambda i: (0, 0))],
      out_specs=pl.BlockSpec((L, blk, HID), lambda i: (0, i, 0)),
      out_shape=jax.ShapeDtypeStruct((L, B, HID), jnp.float32),
  )(ops_t, tab, wih_t, whh_t, bih, bhh)


def _tc_attention_score(gnn_t, gru_t, mask_t, w1_t, b1, w2_t, b2, w3_t, b3,
                        wt_t, bt, item_t):
  blk = 128

  def body(g_ref, r_ref, m_ref, w1_ref, b1_ref, w2_ref, b2_ref, w3_ref,
           b3_ref, wt_ref, bt_ref, item_ref, o_ref):
    last = jnp.concatenate([g_ref[L - 1], r_ref[L - 1]], axis=-1)
    q = jnp.dot(last, w1_ref[...], preferred_element_type=jnp.float32) + b1_ref[0]
    w2 = w2_ref[...]
    b2v = b2_ref[0]
    w3 = w3_ref[...]
    b3v = b3_ref[0, 0]

    def step(t, carry):
      f_t = jnp.concatenate([g_ref[t], r_ref[t]], axis=-1)
      e_t = jax.nn.sigmoid(
          q + jnp.dot(f_t, w2, preferred_element_type=jnp.float32) + b2v)
      a_t = jnp.dot(e_t, w3, preferred_element_type=jnp.float32) + b3v
      a_t = a_t * (1.0 - m_ref[t][:, None])
      return carry + f_t * a_t

    sg = lax.fori_loop(0, L, step, jnp.zeros((blk, 2 * HID), jnp.float32))
    s = (jnp.dot(last, wt_ref[0], preferred_element_type=jnp.float32)
         + jnp.dot(sg, wt_ref[1], preferred_element_type=jnp.float32)
         + bt_ref[0])
    o_ref[...] = jnp.dot(s, item_ref[...], preferred_element_type=jnp.float32)

  return pl.pallas_call(
      body,
      grid=(B // blk,),
      in_specs=[pl.BlockSpec((L, blk, HID), lambda i: (0, i, 0)),
                pl.BlockSpec((L, blk, HID), lambda i: (0, i, 0)),
                pl.BlockSpec((L, blk), lambda i: (0, i)),
                pl.BlockSpec((2 * HID, 2 * HID), lambda i: (0, 0)),
                pl.BlockSpec((1, 2 * HID), lambda i: (0, 0)),
                pl.BlockSpec((2 * HID, 2 * HID), lambda i: (0, 0)),
                pl.BlockSpec((1, 2 * HID), lambda i: (0, 0)),
                pl.BlockSpec((2 * HID, 1), lambda i: (0, 0)),
                pl.BlockSpec((1, 1), lambda i: (0, 0)),
                pl.BlockSpec((NC, 2 * HID, HID), lambda i: (0, 0, 0)),
                pl.BlockSpec((1, HID), lambda i: (0, 0)),
                pl.BlockSpec((HID, N_ITEM), lambda i: (0, 0))],
      out_specs=pl.BlockSpec((blk, N_ITEM), lambda i: (i, 0)),
      out_shape=jax.ShapeDtypeStruct((B, N_ITEM), jnp.float32),
  )(gnn_t, gru_t, mask_t, w1_t, b1, w2_t, b2, w3_t, b3, wt_t, bt, item_t)


def kernel(nodes, edges, edge2seq, ops, mask, entity_table, op_table, gnn_w,
           gnn_wih, gnn_whh, gnn_bih, gnn_bhh, gru_wih, gru_whh, gru_bih,
           gru_bhh, w1, b1, w2, b2, w3, b3, wt, bt):
  f32 = jnp.float32
  nodes_pad = jnp.concatenate(
      [nodes.astype(jnp.int32),
       jnp.arange(10240 - N_NODES, dtype=jnp.int32)])
  ops_t = jnp.transpose(ops).astype(jnp.int32)
  e2s_t = jnp.transpose(edge2seq).reshape(-1).astype(jnp.int32)
  src = edges[0].astype(jnp.int32)
  dst3 = edges[1].astype(jnp.int32).reshape(NW, 125, 80)
  op_tab = jnp.concatenate([op_table, jnp.zeros((4, HID), f32)])
  zeros = jnp.zeros((10240, HID), f32)
  gnn_wih_t = jnp.transpose(gnn_wih)
  gnn_whh_t = jnp.transpose(gnn_whh)
  gru_wih_t = jnp.transpose(gru_wih)
  gru_whh_t = jnp.transpose(gru_whh)
  w1_t = jnp.transpose(w1)
  w2_t = jnp.transpose(w2)
  w3_t = jnp.transpose(w3)
  wt_t = jnp.transpose(wt).reshape(NC, 2 * HID, HID)
  item_t = jnp.transpose(entity_table[:N_ITEM])
  mask_t = jnp.transpose(mask).astype(f32)
  b2d = lambda v: v.reshape(1, -1)

  x = _sc_gather(entity_table, nodes_pad, 80)[:N_NODES]
  for i in range(2):
    m = _tc_matmul(x, gnn_w[i])
    agg2 = _sc_edge_scatter_add(m, src, dst3, zeros)
    x = _tc_gru_cell(agg2, x, gnn_wih_t, gnn_whh_t,
                     b2d(gnn_bih), b2d(gnn_bhh))

  gnn_t = _sc_gather(x, e2s_t, 80).reshape(L, B, HID)

  gru_t = _tc_seq_gru(ops_t, op_tab, gru_wih_t, gru_whh_t,
                      b2d(gru_bih), b2d(gru_bhh))

  return _tc_attention_score(gnn_t, gru_t, mask_t, w1_t, b2d(b1), w2_t,
                             b2d(b2), w3_t, b2d(b3).reshape(1, 1), wt_t,
                             b2d(bt), item_t)


# speedup vs baseline: 6.8945x; 1.2247x over previous
import functools

import jax
import jax.numpy as jnp
from jax import lax
from jax.experimental import pallas as pl
from jax.experimental.pallas import tpu as pltpu
from jax.experimental.pallas import tpu_sc as plsc

HID = 128
N_NODES = 10000
N_EDGES = 320000
N_ITEM = 10000
B = 1024
L = 50
NC = 2
NS = 16
NW = NC * NS

_MESH = plsc.VectorSubcoreMesh(core_axis_name="c", subcore_axis_name="s")


def _sc_gather(table, idx, chunk):
  n_idx = idx.shape[0]
  per_w = n_idx // NW
  n_chunks = per_w // chunk
  assert per_w % chunk == 0 and chunk % 8 == 0 and chunk <= 128
  assert n_chunks % 2 == 0 and n_chunks >= 2

  @functools.partial(
      pl.kernel,
      out_type=jax.ShapeDtypeStruct((n_idx, HID), jnp.float32),
      mesh=_MESH,
      scratch_types=[
          pltpu.VMEM((per_w,), jnp.int32),
          pltpu.VMEM((chunk, HID), jnp.float32),
          pltpu.VMEM((chunk, HID), jnp.float32),
          pltpu.SemaphoreType.DMA,
          pltpu.SemaphoreType.DMA,
      ],
  )
  def k(table_hbm, idx_hbm, out_hbm, idx_v, r0, r1, g0, g1):
    wid = lax.axis_index("c") * NS + lax.axis_index("s")
    base = wid * per_w
    pltpu.sync_copy(idx_hbm.at[pl.ds(base, per_w)], idx_v)

    def gather(i, buf, sem):
      pltpu.async_copy(table_hbm.at[idx_v.at[pl.ds(i * chunk, chunk)]],
                       buf, sem)

    def wait0():
      pltpu.make_async_copy(table_hbm.at[pl.ds(0, chunk)], r0, g0).wait()

    def out(i, buf):
      pltpu.sync_copy(buf, out_hbm.at[pl.ds(base + i * chunk, chunk)])

    gather(0, r0, g0)

    def body(i, carry):
      gather(2 * i + 1, r1, g1)
      wait0()
      out(2 * i, r0)
      gather(2 * i + 2, r0, g0)
      pltpu.make_async_copy(table_hbm.at[pl.ds(0, chunk)], r1, g1).wait()
      out(2 * i + 1, r1)
      return carry

    lax.fori_loop(0, n_chunks // 2 - 1, body, 0)
    gather(n_chunks - 1, r1, g1)
    wait0()
    out(n_chunks - 2, r0)
    pltpu.make_async_copy(table_hbm.at[pl.ds(0, chunk)], r1, g1).wait()
    out(n_chunks - 1, r1)

  return k(table, idx)


def _sc_edge_scatter_add(m, src, dst3, zeros):
  chunk = 80
  per_w = N_EDGES // NW
  n_chunks = per_w // chunk
  n_pad = 10240
  rows_per_tile = n_pad // NS

  @functools.partial(
      pl.kernel,
      out_type=jax.ShapeDtypeStruct((NC, n_pad, HID), jnp.float32),
      mesh=_MESH,
      scratch_types=[
          pltpu.VMEM((per_w,), jnp.int32),
          pltpu.VMEM((n_chunks, chunk), jnp.int32),
          pltpu.VMEM((chunk, HID), jnp.float32),
          pltpu.VMEM((chunk, HID), jnp.float32),
          pltpu.VMEM_SHARED((n_pad, HID), jnp.float32),
          pltpu.SemaphoreType.DMA,
          pltpu.SemaphoreType.DMA,
      ],
  )
  def k(m_hbm, src_hbm, dst_hbm, zero_hbm, out_hbm,
        sidx_v, didx_v, r0, r1, acc_sh, g0, g1):
    c = lax.axis_index("c")
    s = lax.axis_index("s")
    wid = c * NS + s
    base = wid * per_w
    pltpu.sync_copy(src_hbm.at[pl.ds(base, per_w)], sidx_v)
    pltpu.sync_copy(dst_hbm.at[wid], didx_v)
    pltpu.sync_copy(zero_hbm.at[pl.ds(s * rows_per_tile, rows_per_tile)],
                    acc_sh.at[pl.ds(s * rows_per_tile, rows_per_tile)])
    plsc.subcore_barrier()

    def gather(i, buf, sem):
      pltpu.async_copy(m_hbm.at[sidx_v.at[pl.ds(i * chunk, chunk)]], buf, sem)

    def scat(i, buf):
      pltpu.sync_copy(buf, acc_sh.at[didx_v.at[i]], add=True)

    def wait(buf, sem):
      pltpu.make_async_copy(m_hbm.at[pl.ds(0, chunk)], buf, sem).wait()

    gather(0, r0, g0)

    def body(i, carry):
      gather(2 * i + 1, r1, g1)
      wait(r0, g0)
      scat(2 * i, r0)
      gather(2 * i + 2, r0, g0)
      wait(r1, g1)
      scat(2 * i + 1, r1)
      return carry

    lax.fori_loop(0, (n_chunks - 1) // 2, body, 0)
    wait(r0, g0)
    scat(n_chunks - 1, r0)

    plsc.subcore_barrier()
    pltpu.sync_copy(acc_sh.at[pl.ds(s * rows_per_tile, rows_per_tile)],
                    out_hbm.at[c, pl.ds(s * rows_per_tile, rows_per_tile)])

  return k(m, src, dst3, zeros)


def _tc_matmul(x, w):
  n = x.shape[0]
  blk = 1000

  def body(x_ref, w_ref, o_ref):
    o_ref[...] = jnp.dot(x_ref[...], w_ref[...],
                         preferred_element_type=jnp.float32)

  return pl.pallas_call(
      body,
      grid=(n // blk,),
      in_specs=[pl.BlockSpec((blk, HID), lambda i: (i, 0)),
                pl.BlockSpec((HID, HID), lambda i: (0, 0))],
      out_specs=pl.BlockSpec((blk, HID), lambda i: (i, 0)),
      out_shape=jax.ShapeDtypeStruct((n, HID), jnp.float32),
  )(x, w)


def _gates(gi, gh, h):
  i_r, i_z, i_n = gi[:, :HID], gi[:, HID:2 * HID], gi[:, 2 * HID:]
  h_r, h_z, h_n = gh[:, :HID], gh[:, HID:2 * HID], gh[:, 2 * HID:]
  r = jax.nn.sigmoid(i_r + h_r)
  z = jax.nn.sigmoid(i_z + h_z)
  n = jnp.tanh(i_n + r * h_n)
  return (1.0 - z) * n + z * h


def _tc_gru_cell(agg2, x, wih_t, whh_t, bih, bhh):
  n = x.shape[0]
  blk = 1000

  def body(agg_ref, x_ref, wih_ref, whh_ref, bih_ref, bhh_ref, o_ref):
    a = agg_ref[0] + agg_ref[1]
    h = x_ref[...]
    gi = jnp.dot(a, wih_ref[...], preferred_element_type=jnp.float32) + bih_ref[0]
    gh = jnp.dot(h, whh_ref[...], preferred_element_type=jnp.float32) + bhh_ref[0]
    o_ref[...] = _gates(gi, gh, h)

  return pl.pallas_call(
      body,
      grid=(n // blk,),
      in_specs=[pl.BlockSpec((NC, blk, HID), lambda i: (0, i, 0)),
                pl.BlockSpec((blk, HID), lambda i: (i, 0)),
                pl.BlockSpec((HID, 3 * HID), lambda i: (0, 0)),
                pl.BlockSpec((HID, 3 * HID), lambda i: (0, 0)),
                pl.BlockSpec((1, 3 * HID), lambda i: (0, 0)),
                pl.BlockSpec((1, 3 * HID), lambda i: (0, 0))],
      out_specs=pl.BlockSpec((blk, HID), lambda i: (i, 0)),
      out_shape=jax.ShapeDtypeStruct((n, HID), jnp.float32),
  )(agg2, x, wih_t, whh_t, bih, bhh)


def _tc_seq_gru(ops_t, tab, wih_t, whh_t, bih, bhh):
  blk = 1024
  n_op = tab.shape[0]

  def body(ops_ref, tab_ref, wih_ref, whh_ref, bih_ref, bhh_ref, o_ref):
    tab_ih = jnp.dot(tab_ref[...], wih_ref[...],
                     preferred_element_type=jnp.float32)
    whh = whh_ref[...]
    bi = bih_ref[0]
    bh = bhh_ref[0]

    def step(t, h):
      ids = ops_ref[t]
      oh = (ids[:, None] == lax.broadcasted_iota(jnp.int32, (1, n_op), 1)
            ).astype(jnp.float32)
      gi = jnp.dot(oh, tab_ih, preferred_element_type=jnp.float32) + bi
      gh = jnp.dot(h, whh, preferred_element_type=jnp.float32) + bh
      h2 = _gates(gi, gh, h)
      o_ref[t] = h2
      return h2

    lax.fori_loop(0, L, step, jnp.zeros((blk, HID), jnp.float32))

  return pl.pallas_call(
      body,
      grid=(B // blk,),
      in_specs=[pl.BlockSpec((L, blk), lambda i: (0, i)),
                pl.BlockSpec((n_op, HID), lambda i: (0, 0)),
                pl.BlockSpec((HID, 3 * HID), lambda i: (0, 0)),
                pl.BlockSpec((HID, 3 * HID), lambda i: (0, 0)),
                pl.BlockSpec((1, 3 * HID), lambda i: (0, 0)),
                pl.BlockSpec((1, 3 * HID), lambda i: (0, 0))],
      out_specs=pl.BlockSpec((L, blk, HID), lambda i: (0, i, 0)),
      out_shape=jax.ShapeDtypeStruct((L, B, HID), jnp.float32),
  )(ops_t, tab, wih_t, whh_t, bih, bhh)


def _tc_attention_score(gnn_t, gru_t, mask_t, w1_t, b1, w2_t, b2, w3_t, b3,
                        wt_t, bt, item_t):
  blk = 128

  def body(g_ref, r_ref, m_ref, w1_ref, b1_ref, w2_ref, b2_ref, w3_ref,
           b3_ref, wt_ref, bt_ref, item_ref, o_ref):
    dot = functools.partial(jnp.dot, preferred_element_type=jnp.float32)
    last_g = g_ref[L - 1]
    last_r = r_ref[L - 1]
    q = dot(last_g, w1_ref[0]) + dot(last_r, w1_ref[1]) + b1_ref[0]
    g2 = g_ref[...].reshape(L * blk, HID)
    r2 = r_ref[...].reshape(L * blk, HID)
    k = dot(g2, w2_ref[0]) + dot(r2, w2_ref[1]) + b2_ref[0]
    e3 = jax.nn.sigmoid(k.reshape(L, blk, 2 * HID) + q[None])
    alpha = jnp.sum(e3 * w3_ref[0][None, None, :], axis=-1) + b3_ref[0, 0]
    alpha = alpha * (1.0 - m_ref[...])
    aw = alpha[:, :, None]
    sg_g = jnp.sum(g_ref[...] * aw, axis=0)
    sg_r = jnp.sum(r_ref[...] * aw, axis=0)
    s = (dot(last_g, wt_ref[0]) + dot(last_r, wt_ref[1])
         + dot(sg_g, wt_ref[2]) + dot(sg_r, wt_ref[3]) + bt_ref[0])
    o_ref[...] = dot(s, item_ref[...])

  return pl.pallas_call(
      body,
      grid=(B // blk,),
      in_specs=[pl.BlockSpec((L, blk, HID), lambda i: (0, i, 0)),
                pl.BlockSpec((L, blk, HID), lambda i: (0, i, 0)),
                pl.BlockSpec((L, blk), lambda i: (0, i)),
                pl.BlockSpec((2, HID, 2 * HID), lambda i: (0, 0, 0)),
                pl.BlockSpec((1, 2 * HID), lambda i: (0, 0)),
                pl.BlockSpec((2, HID, 2 * HID), lambda i: (0, 0, 0)),
                pl.BlockSpec((1, 2 * HID), lambda i: (0, 0)),
                pl.BlockSpec((1, 2 * HID), lambda i: (0, 0)),
                pl.BlockSpec((1, 1), lambda i: (0, 0)),
                pl.BlockSpec((4, HID, HID), lambda i: (0, 0, 0)),
                pl.BlockSpec((1, HID), lambda i: (0, 0)),
                pl.BlockSpec((HID, N_ITEM), lambda i: (0, 0))],
      out_specs=pl.BlockSpec((blk, N_ITEM), lambda i: (i, 0)),
      out_shape=jax.ShapeDtypeStruct((B, N_ITEM), jnp.float32),
  )(gnn_t, gru_t, mask_t, w1_t, b1, w2_t, b2, w3_t, b3, wt_t, bt, item_t)


def kernel(nodes, edges, edge2seq, ops, mask, entity_table, op_table, gnn_w,
           gnn_wih, gnn_whh, gnn_bih, gnn_bhh, gru_wih, gru_whh, gru_bih,
           gru_bhh, w1, b1, w2, b2, w3, b3, wt, bt):
  f32 = jnp.float32
  nodes_pad = jnp.concatenate(
      [nodes.astype(jnp.int32),
       jnp.arange(10240 - N_NODES, dtype=jnp.int32)])
  ops_t = jnp.transpose(ops).astype(jnp.int32)
  e2s_t = jnp.transpose(edge2seq).reshape(-1).astype(jnp.int32)
  src = edges[0].astype(jnp.int32)
  dst3 = edges[1].astype(jnp.int32).reshape(NW, 125, 80)
  op_tab = jnp.concatenate([op_table, jnp.zeros((4, HID), f32)])
  zeros = jnp.zeros((10240, HID), f32)
  gnn_wih_t = jnp.transpose(gnn_wih)
  gnn_whh_t = jnp.transpose(gnn_whh)
  gru_wih_t = jnp.transpose(gru_wih)
  gru_whh_t = jnp.transpose(gru_whh)
  w1_t = jnp.transpose(w1).reshape(2, HID, 2 * HID)
  w2_t = jnp.transpose(w2).reshape(2, HID, 2 * HID)
  w3_t = w3
  wt_t = jnp.transpose(wt).reshape(4, HID, HID)
  item_t = jnp.transpose(entity_table[:N_ITEM])
  mask_t = jnp.transpose(mask).astype(f32)
  b2d = lambda v: v.reshape(1, -1)

  x = _sc_gather(entity_table, nodes_pad, 80)[:N_NODES]
  for i in range(2):
    m = _tc_matmul(x, gnn_w[i])
    agg2 = _sc_edge_scatter_add(m, src, dst3, zeros)
    x = _tc_gru_cell(agg2, x, gnn_wih_t, gnn_whh_t,
                     b2d(gnn_bih), b2d(gnn_bhh))

  gnn_t = _sc_gather(x, e2s_t, 80).reshape(L, B, HID)

  gru_t = _tc_seq_gru(ops_t, op_tab, gru_wih_t, gru_whh_t,
                      b2d(gru_bih), b2d(gru_bhh))

  return _tc_attention_score(gnn_t, gru_t, mask_t, w1_t, b2d(b1), w2_t,
                             b2d(b2), w3_t, b2d(b3).reshape(1, 1), wt_t,
                             b2d(bt), item_t)
